# baseline (speedup 1.0000x reference)
import functools

import jax
import jax.numpy as jnp
from jax import lax
from jax.experimental import pallas as pl
from jax.experimental.pallas import tpu as pltpu
from jax.experimental.pallas import tpu_sc as plsc

_DIM = 64
_L = 16
_NW = 32
_BATCH = 16384
_MARGIN = 1.0
_PER_W = _BATCH // _NW
_CHUNK = 128
_NCHUNK = _PER_W // _CHUNK
_G = _PER_W // _L


def _rsqrt(x):
    xi = lax.bitcast_convert_type(x, jnp.int32)
    yi = jnp.int32(0x5F3759DF) - (xi >> 1)
    y = lax.bitcast_convert_type(yi, jnp.float32)
    for _ in range(3):
        y = y * (1.5 - 0.5 * x * y * y)
    return y


def _sc_body(ent_hbm, rel_hbm, heads_hbm, rels_hbm, tails_hbm, out_hbm,
             hidx, ridx, tidx, hrow, rrow, trow, epos, eneg, ostage, sem):
    wid = lax.axis_index("s") * 2 + lax.axis_index("c")
    iota = lax.iota(jnp.int32, _L)

    for pol in range(2):
        rbase = pol * (_BATCH // _CHUNK) + wid * _NCHUNK
        pltpu.sync_copy(heads_hbm.at[pl.ds(rbase, _NCHUNK)], hidx)
        pltpu.sync_copy(rels_hbm.at[pl.ds(rbase, _NCHUNK)], ridx)
        pltpu.sync_copy(tails_hbm.at[pl.ds(rbase, _NCHUNK)], tidx)
        copies = []
        for j in range(_NCHUNK):
            dst = pl.ds(j * _CHUNK, _CHUNK)
            copies.append(pltpu.async_copy(ent_hbm.at[hidx.at[j]], hrow.at[dst], sem))
            copies.append(pltpu.async_copy(rel_hbm.at[ridx.at[j]], rrow.at[dst], sem))
            copies.append(pltpu.async_copy(ent_hbm.at[tidx.at[j]], trow.at[dst], sem))
        for c in copies:
            c.wait()

        eref = epos if pol == 0 else eneg

        def group_body(g, _):
            lanes = g * _L + iota

            def d_body(d, carry):
                hh, rr, tt, hr, ht, rt = carry
                dv = jnp.full((_L,), d, jnp.int32)
                hv = plsc.load_gather(hrow, [lanes, dv])
                rv = plsc.load_gather(rrow, [lanes, dv])
                tv = plsc.load_gather(trow, [lanes, dv])
                return (hh + hv * hv, rr + rv * rv, tt + tv * tv,
                        hr + hv * rv, ht + hv * tv, rt + rv * tv)

            z = jnp.zeros((_L,), jnp.float32)
            hh, rr, tt, hr, ht, rt = lax.fori_loop(0, _DIM, d_body,
                                                   (z, z, z, z, z, z))
            ih = _rsqrt(jnp.maximum(hh, 1e-24))
            ir = _rsqrt(jnp.maximum(rr, 1e-24))
            it = _rsqrt(jnp.maximum(tt, 1e-24))
            e2 = (hh * ih * ih + rr * ir * ir + tt * it * it
                  + 2.0 * (hr * (ih * ir) - ht * (ih * it) - rt * (ir * it)))
            e2 = jnp.maximum(e2, 0.0)
            e = e2 * _rsqrt(jnp.maximum(e2, 1e-30))
            eref[pl.ds(g * _L, _L)] = e
            return 0

        lax.fori_loop(0, _G, group_body, 0)

    def loss_body(g, acc):
        lp = epos[pl.ds(g * _L, _L)]
        ln = eneg[pl.ds(g * _L, _L)]
        return acc + jnp.maximum(_MARGIN + lp - ln, 0.0)

    acc = lax.fori_loop(0, _G, loss_body, jnp.zeros((_L,), jnp.float32))
    ostage[...] = acc
    pltpu.sync_copy(ostage, out_hbm.at[wid])


_sc_call = functools.partial(
    pl.kernel,
    mesh=plsc.VectorSubcoreMesh(core_axis_name="c", subcore_axis_name="s"),
    out_type=jax.ShapeDtypeStruct((_NW, _L), jnp.float32),
    scratch_types=[
        pltpu.VMEM((_NCHUNK, _CHUNK), jnp.int32),
        pltpu.VMEM((_NCHUNK, _CHUNK), jnp.int32),
        pltpu.VMEM((_NCHUNK, _CHUNK), jnp.int32),
        pltpu.VMEM((_PER_W, _DIM), jnp.float32),
        pltpu.VMEM((_PER_W, _DIM), jnp.float32),
        pltpu.VMEM((_PER_W, _DIM), jnp.float32),
        pltpu.VMEM((_PER_W,), jnp.float32),
        pltpu.VMEM((_PER_W,), jnp.float32),
        pltpu.VMEM((_L,), jnp.float32),
        pltpu.SemaphoreType.DMA,
    ],
    compiler_params=pltpu.CompilerParams(needs_layout_passes=False,
                                         use_tc_tiling_on_sc=False),
)(_sc_body)


def kernel(pos_triples, neg_triples, ent_emb, rel_emb):
    tri = jnp.concatenate([pos_triples, neg_triples], axis=0).astype(jnp.int32)
    heads = tri[:, 0].reshape(2 * _BATCH // _CHUNK, _CHUNK)
    rels = tri[:, 1].reshape(2 * _BATCH // _CHUNK, _CHUNK)
    tails = tri[:, 2].reshape(2 * _BATCH // _CHUNK, _CHUNK)
    partials = _sc_call(ent_emb, rel_emb, heads, rels, tails)
    return jnp.sum(partials) / jnp.float32(_BATCH)


# speedup vs baseline: 1.4781x; 1.4781x over previous
import functools

import jax
import jax.numpy as jnp
from jax import lax
from jax.experimental import pallas as pl
from jax.experimental.pallas import tpu as pltpu
from jax.experimental.pallas import tpu_sc as plsc

_DIM = 64
_L = 16
_NW = 32
_BATCH = 16384
_MARGIN = 1.0
_PER_W = _BATCH // _NW
_HALF = _PER_W // 2
_GH = _HALF // _L


def _rsqrt(x):
    xi = lax.bitcast_convert_type(x, jnp.int32)
    yi = jnp.int32(0x5F3759DF) - (xi >> 1)
    y = lax.bitcast_convert_type(yi, jnp.float32)
    for _ in range(3):
        y = y * (1.5 - 0.5 * x * y * y)
    return y


def _sc_body(ent_hbm, rel_hbm, heads_hbm, rels_hbm, tails_hbm, out_hbm,
             hidx_v, ridx_v, tidx_v, hrow, rrow, trow,
             epos, eneg, ostage, sem):
    wid = lax.axis_index("s") * 2 + lax.axis_index("c")
    iota = lax.iota(jnp.int32, _L)

    for pol in range(2):
        base = pol * _BATCH + wid * _PER_W
        pltpu.sync_copy(heads_hbm.at[pl.ds(base, _PER_W)], hidx_v)
        pltpu.sync_copy(rels_hbm.at[pl.ds(base, _PER_W)], ridx_v)
        pltpu.sync_copy(tails_hbm.at[pl.ds(base, _PER_W)], tidx_v)

        eref = epos if pol == 0 else eneg

        for half in range(2):
            hb = half * _HALF

            def issue_body(g, _):
                hvec = hidx_v[pl.ds(hb + g * _L, _L)]
                rvec = ridx_v[pl.ds(hb + g * _L, _L)]
                tvec = tidx_v[pl.ds(hb + g * _L, _L)]
                for j in range(_L):
                    k = g * _L + j
                    pltpu.async_copy(ent_hbm.at[pl.ds(hvec[j], 1), :],
                                     hrow.at[pl.ds(k, 1), :], sem)
                    pltpu.async_copy(rel_hbm.at[pl.ds(rvec[j], 1), :],
                                     rrow.at[pl.ds(k, 1), :], sem)
                    pltpu.async_copy(ent_hbm.at[pl.ds(tvec[j], 1), :],
                                     trow.at[pl.ds(k, 1), :], sem)
                return 0

            lax.fori_loop(0, _GH, issue_body, 0)
            src0 = ent_hbm.at[pl.ds(0, _HALF), :]
            pltpu.make_async_copy(src0, hrow, sem).wait()
            pltpu.make_async_copy(src0, rrow, sem).wait()
            pltpu.make_async_copy(src0, trow, sem).wait()

            def group_body(g, _):
                lanes = g * _L + iota

                def d_body(d, carry):
                    hh, rr, tt, hr, ht, rt = carry
                    dv = jnp.full((_L,), d, jnp.int32)
                    hv = plsc.load_gather(hrow, [lanes, dv])
                    rv = plsc.load_gather(rrow, [lanes, dv])
                    tv = plsc.load_gather(trow, [lanes, dv])
                    return (hh + hv * hv, rr + rv * rv, tt + tv * tv,
                            hr + hv * rv, ht + hv * tv, rt + rv * tv)

                z = jnp.zeros((_L,), jnp.float32)
                hh, rr, tt, hr, ht, rt = lax.fori_loop(0, _DIM, d_body,
                                                       (z, z, z, z, z, z))
                ih = _rsqrt(jnp.maximum(hh, 1e-24))
                ir = _rsqrt(jnp.maximum(rr, 1e-24))
                it = _rsqrt(jnp.maximum(tt, 1e-24))
                e2 = (hh * ih * ih + rr * ir * ir + tt * it * it
                      + 2.0 * (hr * (ih * ir) - ht * (ih * it) - rt * (ir * it)))
                e2 = jnp.maximum(e2, 0.0)
                e = e2 * _rsqrt(jnp.maximum(e2, 1e-30))
                eref[pl.ds(hb + g * _L, _L)] = e
                return 0

            lax.fori_loop(0, _GH, group_body, 0)

    def loss_body(g, acc):
        lp = epos[pl.ds(g * _L, _L)]
        ln = eneg[pl.ds(g * _L, _L)]
        return acc + jnp.maximum(_MARGIN + lp - ln, 0.0)

    acc = lax.fori_loop(0, _PER_W // _L, loss_body,
                        jnp.zeros((_L,), jnp.float32))
    ostage[...] = acc
    pltpu.sync_copy(ostage, out_hbm.at[wid])


_sc_call = functools.partial(
    pl.kernel,
    mesh=plsc.VectorSubcoreMesh(core_axis_name="c", subcore_axis_name="s"),
    out_type=jax.ShapeDtypeStruct((_NW, _L), jnp.float32),
    scratch_types=[
        pltpu.VMEM((_PER_W,), jnp.int32),
        pltpu.VMEM((_PER_W,), jnp.int32),
        pltpu.VMEM((_PER_W,), jnp.int32),
        pltpu.VMEM((_HALF, _DIM), jnp.float32),
        pltpu.VMEM((_HALF, _DIM), jnp.float32),
        pltpu.VMEM((_HALF, _DIM), jnp.float32),
        pltpu.VMEM((_PER_W,), jnp.float32),
        pltpu.VMEM((_PER_W,), jnp.float32),
        pltpu.VMEM((_L,), jnp.float32),
        pltpu.SemaphoreType.DMA,
    ],
    compiler_params=pltpu.CompilerParams(needs_layout_passes=False),
)(_sc_body)


def kernel(pos_triples, neg_triples, ent_emb, rel_emb):
    tri = jnp.concatenate([pos_triples, neg_triples], axis=0).astype(jnp.int32)
    heads = tri[:, 0]
    rels = tri[:, 1]
    tails = tri[:, 2]
    partials = _sc_call(ent_emb, rel_emb, heads, rels, tails)
    return jnp.sum(partials) / jnp.float32(_BATCH)
